# baseline (speedup 1.0000x reference)
import functools

import jax
import jax.numpy as jnp
from jax import lax
from jax.experimental import pallas as pl
from jax.experimental.pallas import tpu as pltpu
from jax.experimental.pallas import tpu_sc as plsc

VOCAB = 1000000
EMBED = 64
BATCH = 16384
HIST = 50

NC, NS = 2, 16
NW = NC * NS
NBUF = 2

WCH = 128
NFULL = VOCAB // WCH
T_PER_W = -(-(NFULL) // NW)

B = BATCH * HIST
CHUNK = 256
CPH = BATCH // CHUNK
NCHUNK = B // CHUNK
G_PER_W = NCHUNK // NW


def _make_transpose():
    mesh = plsc.VectorSubcoreMesh(core_axis_name="c", subcore_axis_name="s")

    @functools.partial(
        pl.kernel,
        mesh=mesh,
        out_type=jax.ShapeDtypeStruct((VOCAB // 2, 2 * EMBED), jnp.float32),
        compiler_params=pltpu.CompilerParams(use_tc_tiling_on_sc=True,
                                             needs_layout_passes=False),
        scratch_types=(
            [pltpu.VMEM((EMBED, WCH), jnp.float32) for _ in range(NBUF)]
            + [pltpu.VMEM((EMBED, WCH), jnp.float32) for _ in range(NBUF)]
            + [pltpu.SemaphoreType.DMA for _ in range(NBUF)]
            + [pltpu.VMEM((EMBED, 64), jnp.float32)]
        ),
    )
    def tr_kernel(tblt_hbm, out_hbm, vin0, vin1, vout0, vout1, wsem0, wsem1,
                  vrem):
        wid = lax.axis_index("s") * NC + lax.axis_index("c")
        vin = [vin0, vin1]
        vout = [vout0, vout1]
        wsems = [wsem0, wsem1]
        iota16 = lax.iota(jnp.int32, 16)

        def process(i, s, first):
            c = wid + i * NW

            @pl.when(c < NFULL)
            def _():
                w0 = pl.multiple_of(c * WCH, WCH)
                pltpu.sync_copy(tblt_hbm.at[:, pl.ds(w0, WCH)], vin[s])
                if not first:
                    pltpu.make_async_copy(
                        vout[s], out_hbm.at[pl.ds(0, EMBED)], wsems[s]).wait()

                def rbody(r, carry):
                    for t in range(8):
                        col = jnp.full((16,), 2 * r + (1 if t >= 4 else 0),
                                       jnp.int32)
                        v = plsc.load_gather(vin[s],
                                             [iota16 + 16 * (t % 4), col])
                        vout[s][r, pl.ds(16 * t, 16)] = v
                    return carry

                lax.fori_loop(0, EMBED, rbody, 0)
                pltpu.async_copy(
                    vout[s],
                    out_hbm.at[pl.ds(pl.multiple_of(w0 // 2, EMBED), EMBED)],
                    wsems[s])

        for i in range(2):
            process(i, i, True)

        def body(i, carry):
            for s in range(NBUF):
                @pl.when((i % NBUF) == s)
                def _():
                    process(i, s, False)
            return carry

        lax.fori_loop(2, T_PER_W, body, 0)
        for s in range(NBUF):
            pltpu.make_async_copy(vout[s], out_hbm.at[pl.ds(0, EMBED)],
                                  wsems[s]).wait()

        @pl.when(wid == 0)
        def _():
            w0 = NFULL * WCH
            pltpu.sync_copy(tblt_hbm.at[:, pl.ds(w0, 64)], vrem)

            def rbody2(r, carry):
                for t in range(8):
                    col = jnp.full((16,), 2 * r + (1 if t >= 4 else 0),
                                   jnp.int32)
                    v = plsc.load_gather(vrem, [iota16 + 16 * (t % 4), col])
                    vout0[r, pl.ds(16 * t, 16)] = v
                return carry

            lax.fori_loop(0, 32, rbody2, 0)
            pltpu.sync_copy(vout0.at[pl.ds(0, 32)],
                            out_hbm.at[pl.ds(w0 // 2, 32)])

    return tr_kernel


def _make_gather():
    mesh = plsc.VectorSubcoreMesh(core_axis_name="c", subcore_axis_name="s")

    @functools.partial(
        pl.kernel,
        mesh=mesh,
        out_type=jax.ShapeDtypeStruct((HIST, EMBED, BATCH), jnp.float32),
        compiler_params=pltpu.CompilerParams(use_tc_tiling_on_sc=False,
                                             needs_layout_passes=False),
        scratch_types=(
            [pltpu.VMEM((CHUNK,), jnp.int32) for _ in range(NBUF)]
            + [pltpu.VMEM((CHUNK, EMBED), jnp.float32) for _ in range(NBUF)]
            + [pltpu.VMEM((EMBED, CHUNK), jnp.float32) for _ in range(NBUF)]
            + [pltpu.SemaphoreType.DMA for _ in range(NBUF)]
            + [pltpu.SemaphoreType.DMA for _ in range(NBUF)]
        ),
    )
    def gather_kernel(idx_hbm, table_hbm, out_hbm,
                      idx0, idx1, rows0, rows1, colt0, colt1,
                      gsem0, gsem1, wsem0, wsem1):
        wid = lax.axis_index("s") * NC + lax.axis_index("c")
        idx_v = [idx0, idx1]
        rows_v = [rows0, rows1]
        colt_v = [colt0, colt1]
        gsems = [gsem0, gsem1]
        wsems = [wsem0, wsem1]
        iota16 = lax.iota(jnp.int32, 16)

        def start(i, s):
            c = wid + i * NW
            off = c * CHUNK
            pltpu.sync_copy(idx_hbm.at[pl.ds(off, CHUNK)], idx_v[s])
            pltpu.async_copy(table_hbm.at[idx_v[s]], rows_v[s], gsems[s])

        def drain(i, s, first):
            c = wid + i * NW
            h = c // CPH
            b0 = (c % CPH) * CHUNK
            pltpu.make_async_copy(table_hbm.at[idx_v[s]], rows_v[s],
                                  gsems[s]).wait()
            if not first:
                pltpu.make_async_copy(colt_v[s], out_hbm.at[0, :,
                                      pl.ds(0, CHUNK)], wsems[s]).wait()

            def jbody(j, carry):
                rowi = iota16 + j * 16
                for e in range(EMBED):
                    cole = jnp.full((16,), e, jnp.int32)
                    v = plsc.load_gather(rows_v[s], [rowi, cole])
                    colt_v[s][e, pl.ds(j * 16, 16)] = v
                return carry

            lax.fori_loop(0, CHUNK // 16, jbody, 0)
            pltpu.async_copy(colt_v[s], out_hbm.at[h, :, pl.ds(b0, CHUNK)],
                             wsems[s])

        start(0, 0)
        start(1, 1)
        drain(0, 0, True)
        start(2, 0)
        drain(1, 1, True)

        def body(i, carry):
            for s in range(NBUF):
                @pl.when((i % NBUF) == s)
                def _():
                    start(i, s)
            for s in range(NBUF):
                @pl.when(((i - 1) % NBUF) == s)
                def _():
                    drain(i - 1, s, False)
            return carry

        lax.fori_loop(3, G_PER_W, body, 0)
        for s in range(NBUF):
            @pl.when(((G_PER_W - 1) % NBUF) == s)
            def _():
                drain(G_PER_W - 1, s, False)
        for s in range(NBUF):
            pltpu.make_async_copy(colt_v[s], out_hbm.at[0, :,
                                  pl.ds(0, CHUNK)], wsems[s]).wait()

    return gather_kernel


_transpose = _make_transpose()
_gather = _make_gather()


def kernel(data, iword_indicator, iword_numerals, ivectors_weight):
    tbl_c = _transpose(ivectors_weight.T)
    tbl = tbl_c.reshape(VOCAB, EMBED)
    idx = data.T.reshape(-1).astype(jnp.int32)
    out3 = _gather(idx, tbl)
    embed = out3.transpose(2, 0, 1)
    if iword_numerals.shape[0] == 0:
        return embed
    numerals = jnp.sign(iword_numerals) * jnp.log(jnp.abs(iword_numerals) + 1.0)
    ne = jnp.ones((EMBED, numerals.shape[0]), jnp.float32).at[0].set(numerals)
    ne = ne.T / (EMBED * 2)
    flat2 = embed.reshape(-1, EMBED)
    mask = iword_indicator.reshape(-1)
    pos = jnp.nonzero(mask, size=iword_numerals.shape[0])[0]
    return flat2.at[pos].set(ne).reshape(embed.shape)


# speedup vs baseline: 1.7254x; 1.7254x over previous
import functools

import jax
import jax.numpy as jnp
from jax import lax
from jax.experimental import pallas as pl
from jax.experimental.pallas import tpu as pltpu
from jax.experimental.pallas import tpu_sc as plsc

VOCAB = 1000000
EMBED = 64
BATCH = 16384
HIST = 50

B = BATCH * HIST
NC, NS = 2, 16
NW = NC * NS
CHUNK = 256
CPH = BATCH // CHUNK
NCHUNK = B // CHUNK
G_PER_W = NCHUNK // NW
NBUF = 2
LG = CHUNK // 16


def _make_gather():
    mesh = plsc.VectorSubcoreMesh(core_axis_name="c", subcore_axis_name="s")

    @functools.partial(
        pl.kernel,
        mesh=mesh,
        out_type=jax.ShapeDtypeStruct((HIST, 8, BATCH // 128, 8, 128),
                                      jnp.float32),
        compiler_params=pltpu.CompilerParams(use_tc_tiling_on_sc=False,
                                             needs_layout_passes=False,
                                             disable_bounds_checks=True),
        scratch_types=(
            [pltpu.VMEM((CHUNK,), jnp.int32) for _ in range(NBUF)]
            + [pltpu.VMEM((CHUNK, EMBED), jnp.float32) for _ in range(NBUF)]
            + [pltpu.VMEM((8, CHUNK // 128, 8, 128), jnp.float32)
               for _ in range(NBUF)]
            + [pltpu.SemaphoreType.DMA for _ in range(NBUF)]
            + [pltpu.SemaphoreType.DMA for _ in range(NBUF)]
        ),
    )
    def gather_kernel(idx_hbm, table_hbm, out_hbm,
                      idx0, idx1, rows0, rows1, colt0, colt1,
                      gsem0, gsem1, wsem0, wsem1):
        wid = lax.axis_index("s") * NC + lax.axis_index("c")
        idx_v = [idx0, idx1]
        rows_v = [rows0, rows1]
        colt_v = [colt0, colt1]
        gsems = [gsem0, gsem1]
        wsems = [wsem0, wsem1]
        iota16 = lax.iota(jnp.int32, 16)
        rowi = [iota16 + 16 * j for j in range(LG)]

        def start(i, s):
            c = wid + i * NW
            off = c * CHUNK
            pltpu.sync_copy(idx_hbm.at[pl.ds(off, CHUNK)], idx_v[s])
            pltpu.async_copy(table_hbm.at[idx_v[s]], rows_v[s], gsems[s])

        def drain(i, s, first):
            c = wid + i * NW
            h = c // CPH
            b0 = (c % CPH) * CHUNK
            pltpu.make_async_copy(table_hbm.at[idx_v[s]], rows_v[s],
                                  gsems[s]).wait()
            if not first:
                pltpu.make_async_copy(colt_v[s], out_hbm.at[0, :,
                                      pl.ds(0, CHUNK // 128)], wsems[s]).wait()

            def ebody(e, carry):
                cole = jnp.zeros((16,), jnp.int32) + e
                for j in range(LG):
                    v = plsc.load_gather(rows_v[s], [rowi[j], cole])
                    colt_v[s][e // 8, j // 8, e % 8,
                              pl.ds(16 * (j % 8), 16)] = v
                return carry

            lax.fori_loop(0, EMBED, ebody, 0)
            pltpu.async_copy(colt_v[s],
                             out_hbm.at[h, :, pl.ds(b0 // 128, CHUNK // 128)],
                             wsems[s])

        start(0, 0)
        start(1, 1)
        drain(0, 0, True)
        start(2, 0)
        drain(1, 1, True)

        def body(i, carry):
            for s in range(NBUF):
                @pl.when((i % NBUF) == s)
                def _():
                    start(i, s)
            for s in range(NBUF):
                @pl.when(((i - 1) % NBUF) == s)
                def _():
                    drain(i - 1, s, False)
            return carry

        lax.fori_loop(3, G_PER_W, body, 0)
        for s in range(NBUF):
            @pl.when(((G_PER_W - 1) % NBUF) == s)
            def _():
                drain(G_PER_W - 1, s, False)
        for s in range(NBUF):
            pltpu.make_async_copy(colt_v[s], out_hbm.at[0, :,
                                  pl.ds(0, CHUNK // 128)], wsems[s]).wait()

    return gather_kernel


_gather = _make_gather()


def kernel(data, iword_indicator, iword_numerals, ivectors_weight):
    idx = data.T.reshape(-1).astype(jnp.int32)
    out5 = _gather(idx, ivectors_weight)
    embed = out5.transpose(2, 4, 0, 1, 3).reshape(BATCH, HIST, EMBED)
    if iword_numerals.shape[0] == 0:
        return embed
    numerals = jnp.sign(iword_numerals) * jnp.log(jnp.abs(iword_numerals) + 1.0)
    ne = jnp.ones((EMBED, numerals.shape[0]), jnp.float32).at[0].set(numerals)
    ne = ne.T / (EMBED * 2)
    flat2 = embed.reshape(-1, EMBED)
    mask = iword_indicator.reshape(-1)
    pos = jnp.nonzero(mask, size=iword_numerals.shape[0])[0]
    return flat2.at[pos].set(ne).reshape(embed.shape)
